# Initial kernel scaffold; baseline (speedup 1.0000x reference)
import functools

import jax
import jax.numpy as jnp
from jax import lax
from jax.experimental import pallas as pl
from jax.experimental.pallas import tpu as pltpu
from jax.experimental.pallas import tpu_sc as plsc

N = 10000
E = 320000
D_IN = 128
F = 64
OUT = 10
G = 64

NC = 2
NS = 16
NP = 10240
RPT = NP // NS
EPT = E // NS
CHUNK = 80
NCHUNK = EPT // CHUNK
HALF = NCHUNK // NC


def _tc_prep1_body(x_ref, w_ref, as_ref, ad_ref, h_ref, asr_ref, adr_ref, m_ref):
    h = jnp.dot(x_ref[...], w_ref[...], preferred_element_type=jnp.float32)
    h_ref[...] = h
    asr = jnp.sum(h * as_ref[...], axis=1, keepdims=True)
    adr = jnp.sum(h * ad_ref[...], axis=1, keepdims=True)
    asr_ref[...] = asr
    adr_ref[...] = adr
    m = jnp.maximum(jnp.max(asr) + jnp.max(adr), 0.0)
    m_ref[...] = jnp.full((8, 128), m, jnp.float32)


def _tc_prep2_body(o0_ref, o1_ref, b_ref, w_ref, as_ref, ad_ref,
                   xl_ref, h_ref, asr_ref, adr_ref, m_ref):
    xl = jnp.maximum(o0_ref[...] + o1_ref[...] + b_ref[...], 0.0)
    xl_ref[...] = xl
    h = jnp.dot(xl, w_ref[...], preferred_element_type=jnp.float32)
    h_ref[...] = h
    asr = jnp.sum(h * as_ref[...], axis=1, keepdims=True)
    adr = jnp.sum(h * ad_ref[...], axis=1, keepdims=True)
    asr_ref[...] = asr
    adr_ref[...] = adr
    m = jnp.maximum(jnp.max(asr) + jnp.max(adr), 0.0)
    m_ref[...] = jnp.full((8, 128), m, jnp.float32)


def _tc_final_body(x1_ref, x2_ref, o0_ref, o1_ref, b3_ref, batch_ref,
                   wl_ref, bl_ref, out_ref):
    x3 = jnp.maximum(o0_ref[...] + o1_ref[...] + b3_ref[...], 0.0)
    xc = jnp.concatenate([x1_ref[...], x2_ref[...], x3], axis=1)
    gid = lax.broadcasted_iota(jnp.int32, (N, G), 1)
    oh = (batch_ref[...] == gid).astype(jnp.float32)
    sums = lax.dot_general(oh, xc, (((0,), (0,)), ((), ())),
                           preferred_element_type=jnp.float32)
    ones = jnp.ones((N, 1), jnp.float32)
    counts = lax.dot_general(oh, ones, (((0,), (0,)), ((), ())),
                             preferred_element_type=jnp.float32)
    pooled = sums / jnp.maximum(counts, 1.0)
    logits = jnp.dot(pooled, wl_ref[...],
                     preferred_element_type=jnp.float32) + bl_ref[...]
    z = logits - jnp.max(logits, axis=1, keepdims=True)
    ez = jnp.exp(z)
    out_ref[...] = ez / jnp.sum(ez, axis=1, keepdims=True)


def _tc_prep1(x, w, a_s, a_d):
    return pl.pallas_call(
        _tc_prep1_body,
        out_shape=[
            jax.ShapeDtypeStruct((N, F), jnp.float32),
            jax.ShapeDtypeStruct((N, 1), jnp.float32),
            jax.ShapeDtypeStruct((N, 1), jnp.float32),
            jax.ShapeDtypeStruct((8, 128), jnp.float32),
        ],
    )(x, w, a_s.reshape(1, F), a_d.reshape(1, F))


def _tc_prep2(o0, o1, b, w, a_s, a_d):
    return pl.pallas_call(
        _tc_prep2_body,
        out_shape=[
            jax.ShapeDtypeStruct((N, F), jnp.float32),
            jax.ShapeDtypeStruct((N, F), jnp.float32),
            jax.ShapeDtypeStruct((N, 1), jnp.float32),
            jax.ShapeDtypeStruct((N, 1), jnp.float32),
            jax.ShapeDtypeStruct((8, 128), jnp.float32),
        ],
    )(o0, o1, b.reshape(1, F), w, a_s.reshape(1, F), a_d.reshape(1, F))


def _tc_final(x1, x2, o0, o1, b3, batch, wl, bl):
    return pl.pallas_call(
        _tc_final_body,
        out_shape=jax.ShapeDtypeStruct((G, OUT), jnp.float32),
    )(x1, x2, o0, o1, b3.reshape(1, F), batch.reshape(N, 1), wl,
      bl.reshape(1, OUT))


_SC_MESH = plsc.VectorSubcoreMesh(core_axis_name="c", subcore_axis_name="s")


@functools.partial(
    pl.kernel,
    out_type=jax.ShapeDtypeStruct((NC, NP, F), jnp.float32),
    mesh=_SC_MESH,
    scratch_types=[
        pltpu.VMEM((NCHUNK, CHUNK), jnp.int32),
        pltpu.VMEM((NCHUNK, CHUNK), jnp.int32),
        pltpu.VMEM((NCHUNK, CHUNK), jnp.float32),
        pltpu.VMEM((N,), jnp.float32),
        pltpu.VMEM((N,), jnp.float32),
        pltpu.VMEM((NP,), jnp.float32),
        pltpu.VMEM((CHUNK, F), jnp.float32),
        pltpu.VMEM((CHUNK,), jnp.float32),
        pltpu.VMEM((16,), jnp.float32),
        pltpu.VMEM_SHARED((NP,), jnp.float32),
        pltpu.VMEM_SHARED((NP, F), jnp.float32),
        pltpu.SemaphoreType.DMA,
    ],
)
def _sc_layer(h_hbm, as_hbm, ad_hbm, m_hbm, src_hbm, dst_hbm, out_hbm,
              src_v, dst_v, w_v, as_v, ad_v, denom_v, rowbuf, alpha_v, m_v,
              denom_sh, out_sh, sem):
    sid = lax.axis_index("s")
    cid = lax.axis_index("c")

    pltpu.sync_copy(src_hbm.at[sid], src_v)
    pltpu.sync_copy(dst_hbm.at[sid], dst_v)
    pltpu.sync_copy(as_hbm, as_v)
    pltpu.sync_copy(ad_hbm, ad_v)
    pltpu.sync_copy(m_hbm.at[pl.ds(0, 16)], m_v)

    zero16 = jnp.zeros((16,), jnp.float32)

    def _zrow(r, carry):
        for c in range(F // 16):
            rowbuf[r, pl.ds(c * 16, 16)] = zero16
        return carry
    lax.fori_loop(0, CHUNK, _zrow, 0)
    for k in range(RPT // CHUNK):
        pltpu.sync_copy(rowbuf, out_sh.at[pl.ds(sid * RPT + k * CHUNK, CHUNK)])

    def _zden(r, carry):
        denom_v[pl.ds(r * 16, 16)] = zero16
        return carry
    lax.fori_loop(0, RPT // 16, _zden, 0)
    pltpu.sync_copy(denom_v.at[pl.ds(0, RPT)], denom_sh.at[pl.ds(sid * RPT, RPT)])

    m_vec = m_v[...]

    def _p1(j, carry):
        for g in range(CHUNK // 16):
            sv = src_v[j, pl.ds(g * 16, 16)]
            dv = dst_v[j, pl.ds(g * 16, 16)]
            e = plsc.load_gather(as_v, [sv]) + plsc.load_gather(ad_v, [dv])
            e = jnp.where(e >= 0.0, e, e * 0.2)
            w_v[j, pl.ds(g * 16, 16)] = jnp.exp(e - m_vec)
        return carry
    lax.fori_loop(0, NCHUNK, _p1, 0)

    plsc.subcore_barrier()

    def _p1s(j, carry):
        pltpu.sync_copy(w_v.at[j], denom_sh.at[dst_v.at[j]], add=True)
        return carry
    lax.fori_loop(0, NCHUNK, _p1s, 0)

    plsc.subcore_barrier()

    pltpu.sync_copy(denom_sh, denom_v)

    base = cid * HALF

    def _p2(jj, carry):
        j = base + jj
        pltpu.async_copy(h_hbm.at[src_v.at[j]], rowbuf, sem).wait()
        for g in range(CHUNK // 16):
            dv = dst_v[j, pl.ds(g * 16, 16)]
            den = plsc.load_gather(denom_v, [dv])
            w = w_v[j, pl.ds(g * 16, 16)]
            alpha_v[pl.ds(g * 16, 16)] = w / den

        def _scale(e, c2):
            a = alpha_v[e]
            for c in range(F // 16):
                rowbuf[e, pl.ds(c * 16, 16)] = rowbuf[e, pl.ds(c * 16, 16)] * a
            return c2
        lax.fori_loop(0, CHUNK, _scale, 0)
        pltpu.sync_copy(rowbuf, out_sh.at[dst_v.at[j]], add=True)
        return carry
    lax.fori_loop(0, HALF, _p2, 0)

    plsc.subcore_barrier()

    pltpu.sync_copy(out_sh.at[pl.ds(sid * RPT, RPT)],
                    out_hbm.at[cid, pl.ds(sid * RPT, RPT)])


def kernel(x, edge_index, batch, W1, a1s, a1d, b1, W2, a2s, a2d, b2,
           W3, a3s, a3d, b3, Wl, bl):
    src3 = edge_index[0].reshape(NS, NCHUNK, CHUNK)
    dst3 = edge_index[1].reshape(NS, NCHUNK, CHUNK)

    h1, as1, ad1, m1 = _tc_prep1(x, W1, a1s, a1d)
    o1 = _sc_layer(h1, as1.reshape(N), ad1.reshape(N), m1[0], src3, dst3)
    x1, h2, as2, ad2, m2 = _tc_prep2(o1[0, :N], o1[1, :N], b1, W2, a2s, a2d)
    o2 = _sc_layer(h2, as2.reshape(N), ad2.reshape(N), m2[0], src3, dst3)
    x2, h3, as3, ad3, m3 = _tc_prep2(o2[0, :N], o2[1, :N], b2, W3, a3s, a3d)
    o3 = _sc_layer(h3, as3.reshape(N), ad3.reshape(N), m3[0], src3, dst3)
    return _tc_final(x1, x2, o3[0, :N], o3[1, :N], b3, batch, Wl, bl)


# speedup vs baseline: 27.1888x; 27.1888x over previous
import functools

import jax
import jax.numpy as jnp
from jax import lax
from jax.experimental import pallas as pl
from jax.experimental.pallas import tpu as pltpu
from jax.experimental.pallas import tpu_sc as plsc

N = 10000
E = 320000
D_IN = 128
F = 64
OUT = 10
G = 64

NC = 2
NS = 16
NP = 10240
RPT = NP // NS
EPT = E // NS
CHUNK = 80
NCHUNK = EPT // CHUNK
HALF = NCHUNK // NC


def _tc_prep1_body(x_ref, w_ref, as_ref, ad_ref, h_ref, asr_ref, adr_ref, m_ref):
    h = jnp.dot(x_ref[...], w_ref[...], preferred_element_type=jnp.float32)
    h_ref[0] = h[:, :F // 2]
    h_ref[1] = h[:, F // 2:]
    asr = jnp.sum(h * as_ref[...], axis=1, keepdims=True)
    adr = jnp.sum(h * ad_ref[...], axis=1, keepdims=True)
    asr_ref[...] = asr
    adr_ref[...] = adr
    m = jnp.maximum(jnp.max(asr) + jnp.max(adr), 0.0)
    m_ref[...] = jnp.full((8, 128), m, jnp.float32)


def _tc_prep2_body(o0_ref, o1_ref, b_ref, w_ref, as_ref, ad_ref,
                   xl_ref, h_ref, asr_ref, adr_ref, m_ref):
    o = jnp.concatenate([o0_ref[...], o1_ref[...]], axis=1)
    xl = jnp.maximum(o + b_ref[...], 0.0)
    xl_ref[...] = xl
    h = jnp.dot(xl, w_ref[...], preferred_element_type=jnp.float32)
    h_ref[0] = h[:, :F // 2]
    h_ref[1] = h[:, F // 2:]
    asr = jnp.sum(h * as_ref[...], axis=1, keepdims=True)
    adr = jnp.sum(h * ad_ref[...], axis=1, keepdims=True)
    asr_ref[...] = asr
    adr_ref[...] = adr
    m = jnp.maximum(jnp.max(asr) + jnp.max(adr), 0.0)
    m_ref[...] = jnp.full((8, 128), m, jnp.float32)


def _tc_final_body(x1_ref, x2_ref, o0_ref, o1_ref, b3_ref, batch_ref,
                   wl_ref, bl_ref, out_ref):
    o = jnp.concatenate([o0_ref[...], o1_ref[...]], axis=1)
    x3 = jnp.maximum(o + b3_ref[...], 0.0)
    xc = jnp.concatenate([x1_ref[...], x2_ref[...], x3], axis=1)
    gid = lax.broadcasted_iota(jnp.int32, (N, G), 1)
    oh = (batch_ref[...] == gid).astype(jnp.float32)
    sums = lax.dot_general(oh, xc, (((0,), (0,)), ((), ())),
                           preferred_element_type=jnp.float32)
    ones = jnp.ones((N, 1), jnp.float32)
    counts = lax.dot_general(oh, ones, (((0,), (0,)), ((), ())),
                             preferred_element_type=jnp.float32)
    pooled = sums / jnp.maximum(counts, 1.0)
    logits = jnp.dot(pooled, wl_ref[...],
                     preferred_element_type=jnp.float32) + bl_ref[...]
    z = logits - jnp.max(logits, axis=1, keepdims=True)
    ez = jnp.exp(z)
    out_ref[...] = ez / jnp.sum(ez, axis=1, keepdims=True)


def _tc_prep1(x, w, a_s, a_d):
    return pl.pallas_call(
        _tc_prep1_body,
        out_shape=[
            jax.ShapeDtypeStruct((NC, N, F // 2), jnp.float32),
            jax.ShapeDtypeStruct((N, 1), jnp.float32),
            jax.ShapeDtypeStruct((N, 1), jnp.float32),
            jax.ShapeDtypeStruct((8, 128), jnp.float32),
        ],
    )(x, w, a_s.reshape(1, F), a_d.reshape(1, F))


def _tc_prep2(o0, o1, b, w, a_s, a_d):
    return pl.pallas_call(
        _tc_prep2_body,
        out_shape=[
            jax.ShapeDtypeStruct((N, F), jnp.float32),
            jax.ShapeDtypeStruct((NC, N, F // 2), jnp.float32),
            jax.ShapeDtypeStruct((N, 1), jnp.float32),
            jax.ShapeDtypeStruct((N, 1), jnp.float32),
            jax.ShapeDtypeStruct((8, 128), jnp.float32),
        ],
    )(o0, o1, b.reshape(1, F), w, a_s.reshape(1, F), a_d.reshape(1, F))


def _tc_final(x1, x2, o0, o1, b3, batch, wl, bl):
    return pl.pallas_call(
        _tc_final_body,
        out_shape=jax.ShapeDtypeStruct((G, OUT), jnp.float32),
    )(x1, x2, o0, o1, b3.reshape(1, F), batch.reshape(N, 1), wl,
      bl.reshape(1, OUT))


_SC_MESH = plsc.VectorSubcoreMesh(core_axis_name="c", subcore_axis_name="s")


@functools.partial(
    pl.kernel,
    out_type=jax.ShapeDtypeStruct((NC, NP, F // 2), jnp.float32),
    mesh=_SC_MESH,
    compiler_params=pltpu.CompilerParams(
        needs_layout_passes=False, use_tc_tiling_on_sc=False),
    scratch_types=[
        pltpu.VMEM((NCHUNK, CHUNK), jnp.int32),
        pltpu.VMEM((NCHUNK, CHUNK), jnp.int32),
        pltpu.VMEM((NCHUNK, CHUNK), jnp.float32),
        pltpu.VMEM((N,), jnp.float32),
        pltpu.VMEM((N,), jnp.float32),
        pltpu.VMEM((NP,), jnp.float32),
        pltpu.VMEM((CHUNK, F // 2), jnp.float32),
        pltpu.VMEM((16,), jnp.float32),
        pltpu.VMEM_SHARED((NP,), jnp.float32),
        pltpu.VMEM_SHARED((NP, F // 2), jnp.float32),
        pltpu.SemaphoreType.DMA,
    ],
)
def _sc_layer(h_hbm, as_hbm, ad_hbm, m_hbm, src_hbm, dst_hbm, out_hbm,
              src_v, dst_v, w_v, as_v, ad_v, denom_v, rowbuf, m_v,
              denom_sh, out_sh, sem):
    sid = lax.axis_index("s")
    cid = lax.axis_index("c")

    pltpu.sync_copy(src_hbm.at[sid], src_v)
    pltpu.sync_copy(dst_hbm.at[sid], dst_v)
    pltpu.sync_copy(as_hbm, as_v)
    pltpu.sync_copy(ad_hbm, ad_v)
    pltpu.sync_copy(m_hbm.at[pl.ds(0, 16)], m_v)

    zero16 = jnp.zeros((16,), jnp.float32)

    def _zrow(r, carry):
        for c in range(F // 32):
            rowbuf[r, pl.ds(c * 16, 16)] = zero16
        return carry
    lax.fori_loop(0, CHUNK, _zrow, 0)
    for k in range(RPT // CHUNK):
        pltpu.sync_copy(rowbuf, out_sh.at[pl.ds(sid * RPT + k * CHUNK, CHUNK)])

    def _zden(r, carry):
        denom_v[pl.ds(r * 16, 16)] = zero16
        return carry
    lax.fori_loop(0, RPT // 16, _zden, 0)
    pltpu.sync_copy(denom_v.at[pl.ds(0, RPT)], denom_sh.at[pl.ds(sid * RPT, RPT)])

    m_vec = m_v[...]

    def _p1(j, carry):
        for g in range(CHUNK // 16):
            sv = src_v[j, pl.ds(g * 16, 16)]
            dv = dst_v[j, pl.ds(g * 16, 16)]
            e = plsc.load_gather(as_v, [sv]) + plsc.load_gather(ad_v, [dv])
            e = jnp.where(e >= 0.0, e, e * 0.2)
            w_v[j, pl.ds(g * 16, 16)] = jnp.exp(e - m_vec)
        return carry
    lax.fori_loop(0, NCHUNK, _p1, 0)

    plsc.subcore_barrier()

    def _p1s(j, carry):
        pltpu.sync_copy(w_v.at[j], denom_sh.at[dst_v.at[j]], add=True)
        return carry
    lax.fori_loop(0, NCHUNK, _p1s, 0)

    plsc.subcore_barrier()

    pltpu.sync_copy(denom_sh, denom_v)

    h_half = h_hbm.at[cid]

    def _p2(j, carry):
        pltpu.async_copy(h_half.at[src_v.at[j]], rowbuf, sem).wait()
        for g in range(CHUNK // 16):
            dv = dst_v[j, pl.ds(g * 16, 16)]
            den = plsc.load_gather(denom_v, [dv])
            w = w_v[j, pl.ds(g * 16, 16)]
            alpha = w / den
            for e in range(16):
                a = alpha[e]
                r = g * 16 + e
                for c in range(F // 32):
                    rowbuf[r, pl.ds(c * 16, 16)] = rowbuf[r, pl.ds(c * 16, 16)] * a
        pltpu.sync_copy(rowbuf, out_sh.at[dst_v.at[j]], add=True)
        return carry
    lax.fori_loop(0, NCHUNK, _p2, 0)

    plsc.subcore_barrier()

    pltpu.sync_copy(out_sh.at[pl.ds(sid * RPT, RPT)],
                    out_hbm.at[cid, pl.ds(sid * RPT, RPT)])


def kernel(x, edge_index, batch, W1, a1s, a1d, b1, W2, a2s, a2d, b2,
           W3, a3s, a3d, b3, Wl, bl):
    src3 = edge_index[0].reshape(NS, NCHUNK, CHUNK)
    dst3 = edge_index[1].reshape(NS, NCHUNK, CHUNK)

    h1, as1, ad1, m1 = _tc_prep1(x, W1, a1s, a1d)
    o1 = _sc_layer(h1, as1.reshape(N), ad1.reshape(N), m1[0], src3, dst3)
    x1, h2, as2, ad2, m2 = _tc_prep2(o1[0, :N], o1[1, :N], b1, W2, a2s, a2d)
    o2 = _sc_layer(h2, as2.reshape(N), ad2.reshape(N), m2[0], src3, dst3)
    x2, h3, as3, ad3, m3 = _tc_prep2(o2[0, :N], o2[1, :N], b2, W3, a3s, a3d)
    o3 = _sc_layer(h3, as3.reshape(N), ad3.reshape(N), m3[0], src3, dst3)
    return _tc_final(x1, x2, o3[0, :N], o3[1, :N], b3, batch, Wl, bl)


# speedup vs baseline: 43.7004x; 1.6073x over previous
import functools

import jax
import jax.numpy as jnp
from jax import lax
from jax.experimental import pallas as pl
from jax.experimental.pallas import tpu as pltpu
from jax.experimental.pallas import tpu_sc as plsc

N = 10000
E = 320000
D_IN = 128
F = 64
OUT = 10
G = 64

NC = 2
NS = 16
NP = 10240
RPT = NP // NS
EPT = E // NS
CHUNK = 80
NCHUNK = EPT // CHUNK
HALF = NCHUNK // NC


def _tc_prep1_body(x_ref, w_ref, as_ref, ad_ref, h_ref, asr_ref, adr_ref, m_ref):
    h = jnp.dot(x_ref[...], w_ref[...], preferred_element_type=jnp.float32)
    h_ref[0] = h[:, :F // 2]
    h_ref[1] = h[:, F // 2:]
    asr = jnp.sum(h * as_ref[...], axis=1, keepdims=True)
    adr = jnp.sum(h * ad_ref[...], axis=1, keepdims=True)
    asr_ref[...] = asr
    adr_ref[...] = adr
    m = jnp.maximum(jnp.max(asr) + jnp.max(adr), 0.0)
    m_ref[...] = jnp.full((8, 128), m, jnp.float32)


def _tc_prep2_body(o0_ref, o1_ref, b_ref, w_ref, as_ref, ad_ref,
                   xl_ref, h_ref, asr_ref, adr_ref, m_ref):
    o = jnp.concatenate([o0_ref[...], o1_ref[...]], axis=1)
    xl = jnp.maximum(o + b_ref[...], 0.0)
    xl_ref[...] = xl
    h = jnp.dot(xl, w_ref[...], preferred_element_type=jnp.float32)
    h_ref[0] = h[:, :F // 2]
    h_ref[1] = h[:, F // 2:]
    asr = jnp.sum(h * as_ref[...], axis=1, keepdims=True)
    adr = jnp.sum(h * ad_ref[...], axis=1, keepdims=True)
    asr_ref[...] = asr
    adr_ref[...] = adr
    m = jnp.maximum(jnp.max(asr) + jnp.max(adr), 0.0)
    m_ref[...] = jnp.full((8, 128), m, jnp.float32)


def _tc_final_body(x1_ref, x2_ref, o0_ref, o1_ref, b3_ref, batch_ref,
                   wl_ref, bl_ref, out_ref):
    o = jnp.concatenate([o0_ref[...], o1_ref[...]], axis=1)
    x3 = jnp.maximum(o + b3_ref[...], 0.0)
    xc = jnp.concatenate([x1_ref[...], x2_ref[...], x3], axis=1)
    gid = lax.broadcasted_iota(jnp.int32, (N, G), 1)
    oh = (batch_ref[...] == gid).astype(jnp.float32)
    sums = lax.dot_general(oh, xc, (((0,), (0,)), ((), ())),
                           preferred_element_type=jnp.float32)
    ones = jnp.ones((N, 1), jnp.float32)
    counts = lax.dot_general(oh, ones, (((0,), (0,)), ((), ())),
                             preferred_element_type=jnp.float32)
    pooled = sums / jnp.maximum(counts, 1.0)
    logits = jnp.dot(pooled, wl_ref[...],
                     preferred_element_type=jnp.float32) + bl_ref[...]
    z = logits - jnp.max(logits, axis=1, keepdims=True)
    ez = jnp.exp(z)
    out_ref[...] = ez / jnp.sum(ez, axis=1, keepdims=True)


def _tc_prep1(x, w, a_s, a_d):
    return pl.pallas_call(
        _tc_prep1_body,
        out_shape=[
            jax.ShapeDtypeStruct((NC, N, F // 2), jnp.float32),
            jax.ShapeDtypeStruct((N, 1), jnp.float32),
            jax.ShapeDtypeStruct((N, 1), jnp.float32),
            jax.ShapeDtypeStruct((8, 128), jnp.float32),
        ],
    )(x, w, a_s.reshape(1, F), a_d.reshape(1, F))


def _tc_prep2(o0, o1, b, w, a_s, a_d):
    return pl.pallas_call(
        _tc_prep2_body,
        out_shape=[
            jax.ShapeDtypeStruct((N, F), jnp.float32),
            jax.ShapeDtypeStruct((NC, N, F // 2), jnp.float32),
            jax.ShapeDtypeStruct((N, 1), jnp.float32),
            jax.ShapeDtypeStruct((N, 1), jnp.float32),
            jax.ShapeDtypeStruct((8, 128), jnp.float32),
        ],
    )(o0, o1, b.reshape(1, F), w, a_s.reshape(1, F), a_d.reshape(1, F))


def _tc_final(x1, x2, o0, o1, b3, batch, wl, bl):
    return pl.pallas_call(
        _tc_final_body,
        out_shape=jax.ShapeDtypeStruct((G, OUT), jnp.float32),
    )(x1, x2, o0, o1, b3.reshape(1, F), batch.reshape(N, 1), wl,
      bl.reshape(1, OUT))


_SC_MESH = plsc.VectorSubcoreMesh(core_axis_name="c", subcore_axis_name="s")


@functools.partial(
    pl.kernel,
    out_type=jax.ShapeDtypeStruct((NC, NP, F // 2), jnp.float32),
    mesh=_SC_MESH,
    compiler_params=pltpu.CompilerParams(
        needs_layout_passes=False, use_tc_tiling_on_sc=False),
    scratch_types=[
        pltpu.VMEM((NCHUNK, CHUNK), jnp.int32),
        pltpu.VMEM((NCHUNK, CHUNK), jnp.int32),
        pltpu.VMEM((NCHUNK, CHUNK), jnp.float32),
        pltpu.VMEM((N,), jnp.float32),
        pltpu.VMEM((N,), jnp.float32),
        pltpu.VMEM((NP,), jnp.float32),
        pltpu.VMEM((CHUNK, F // 2), jnp.float32),
        pltpu.VMEM((CHUNK, F // 2), jnp.float32),
        pltpu.VMEM((16,), jnp.float32),
        pltpu.VMEM_SHARED((NP,), jnp.float32),
        pltpu.VMEM_SHARED((NP, F // 2), jnp.float32),
        pltpu.SemaphoreType.DMA,
        pltpu.SemaphoreType.DMA,
        pltpu.SemaphoreType.DMA,
    ],
)
def _sc_layer(h_hbm, as_hbm, ad_hbm, m_hbm, src_hbm, dst_hbm, out_hbm,
              src_v, dst_v, w_v, as_v, ad_v, denom_v, rowbuf0, rowbuf1, m_v,
              denom_sh, out_sh, sem, sem0, sem1):
    sid = lax.axis_index("s")
    cid = lax.axis_index("c")

    pltpu.sync_copy(src_hbm.at[sid], src_v)
    pltpu.sync_copy(dst_hbm.at[sid], dst_v)
    pltpu.sync_copy(as_hbm, as_v)
    pltpu.sync_copy(ad_hbm, ad_v)
    pltpu.sync_copy(m_hbm.at[pl.ds(0, 16)], m_v)

    zero16 = jnp.zeros((16,), jnp.float32)

    def _zrow(r, carry):
        for c in range(F // 32):
            rowbuf0[r, pl.ds(c * 16, 16)] = zero16
        return carry
    lax.fori_loop(0, CHUNK, _zrow, 0)
    for k in range(RPT // CHUNK):
        pltpu.sync_copy(rowbuf0, out_sh.at[pl.ds(sid * RPT + k * CHUNK, CHUNK)])

    def _zden(r, carry):
        denom_v[pl.ds(r * 16, 16)] = zero16
        return carry
    lax.fori_loop(0, RPT // 16, _zden, 0)
    pltpu.sync_copy(denom_v.at[pl.ds(0, RPT)], denom_sh.at[pl.ds(sid * RPT, RPT)])

    plsc.subcore_barrier()

    m_vec = m_v[...]

    def _p1(j, carry):
        for g in range(CHUNK // 16):
            sv = src_v[j, pl.ds(g * 16, 16)]
            dv = dst_v[j, pl.ds(g * 16, 16)]
            e = plsc.load_gather(as_v, [sv]) + plsc.load_gather(ad_v, [dv])
            e = jnp.where(e >= 0.0, e, e * 0.2)
            w_v[j, pl.ds(g * 16, 16)] = jnp.exp(e - m_vec)
        pltpu.async_copy(w_v.at[j], denom_sh.at[dst_v.at[j]], sem, add=True)
        return carry
    lax.fori_loop(0, NCHUNK, _p1, 0)

    def _p1d(j, carry):
        pltpu.make_async_copy(w_v.at[j], denom_sh.at[dst_v.at[j]], sem).wait()
        return carry
    lax.fori_loop(0, NCHUNK, _p1d, 0)

    plsc.subcore_barrier()

    pltpu.sync_copy(denom_sh, denom_v)
    one16 = jnp.ones((16,), jnp.float32)

    def _rcp(r, carry):
        d = denom_v[pl.ds(r * 16, 16)]
        denom_v[pl.ds(r * 16, 16)] = one16 / jnp.maximum(d, 1e-30)
        return carry
    lax.fori_loop(0, NP // 16, _rcp, 0)

    h_half = h_hbm.at[cid]

    def _gather(j, buf, s):
        return pltpu.make_async_copy(h_half.at[src_v.at[j]], buf, s)

    def _proc(j, buf):
        for g in range(CHUNK // 16):
            dv = dst_v[j, pl.ds(g * 16, 16)]
            rden = plsc.load_gather(denom_v, [dv])
            w = w_v[j, pl.ds(g * 16, 16)]
            alpha = w * rden
            for e in range(16):
                a = alpha[e]
                r = g * 16 + e
                for c in range(F // 32):
                    buf[r, pl.ds(c * 16, 16)] = buf[r, pl.ds(c * 16, 16)] * a
        pltpu.sync_copy(buf, out_sh.at[dst_v.at[j]], add=True)

    _gather(0, rowbuf0, sem0).start()

    def _p2(t, carry):
        j0 = 2 * t
        _gather(j0 + 1, rowbuf1, sem1).start()
        _gather(j0, rowbuf0, sem0).wait()
        _proc(j0, rowbuf0)

        @pl.when(t < NCHUNK // 2 - 1)
        def _():
            _gather(j0 + 2, rowbuf0, sem0).start()

        _gather(j0 + 1, rowbuf1, sem1).wait()
        _proc(j0 + 1, rowbuf1)
        return carry
    lax.fori_loop(0, NCHUNK // 2, _p2, 0)

    plsc.subcore_barrier()

    pltpu.sync_copy(out_sh.at[pl.ds(sid * RPT, RPT)],
                    out_hbm.at[cid, pl.ds(sid * RPT, RPT)])


def kernel(x, edge_index, batch, W1, a1s, a1d, b1, W2, a2s, a2d, b2,
           W3, a3s, a3d, b3, Wl, bl):
    src3 = edge_index[0].reshape(NS, NCHUNK, CHUNK)
    dst3 = edge_index[1].reshape(NS, NCHUNK, CHUNK)

    h1, as1, ad1, m1 = _tc_prep1(x, W1, a1s, a1d)
    o1 = _sc_layer(h1, as1.reshape(N), ad1.reshape(N), m1[0], src3, dst3)
    x1, h2, as2, ad2, m2 = _tc_prep2(o1[0, :N], o1[1, :N], b1, W2, a2s, a2d)
    o2 = _sc_layer(h2, as2.reshape(N), ad2.reshape(N), m2[0], src3, dst3)
    x2, h3, as3, ad3, m3 = _tc_prep2(o2[0, :N], o2[1, :N], b2, W3, a3s, a3d)
    o3 = _sc_layer(h3, as3.reshape(N), ad3.reshape(N), m3[0], src3, dst3)
    return _tc_final(x1, x2, o3[0, :N], o3[1, :N], b3, batch, Wl, bl)


# speedup vs baseline: 45.5268x; 1.0418x over previous
import functools

import jax
import jax.numpy as jnp
from jax import lax
from jax.experimental import pallas as pl
from jax.experimental.pallas import tpu as pltpu
from jax.experimental.pallas import tpu_sc as plsc

N = 10000
E = 320000
D_IN = 128
F = 64
OUT = 10
G = 64

NC = 2
NS = 16
NP = 10240
RPT = NP // NS
EPT = E // NS
CHUNK = 80
NCHUNK = EPT // CHUNK
HALF = NCHUNK // NC


def _tc_prep1_body(x_ref, w_ref, as_ref, ad_ref, h_ref, asr_ref, adr_ref, m_ref):
    h = jnp.dot(x_ref[...], w_ref[...], preferred_element_type=jnp.float32)
    h_ref[0] = h[:, :F // 2]
    h_ref[1] = h[:, F // 2:]
    asr = jnp.sum(h * as_ref[...], axis=1, keepdims=True)
    adr = jnp.sum(h * ad_ref[...], axis=1, keepdims=True)
    asr_ref[...] = asr
    adr_ref[...] = adr
    m = jnp.maximum(jnp.max(asr) + jnp.max(adr), 0.0)
    m_ref[...] = jnp.full((8, 128), m, jnp.float32)


def _tc_prep2_body(o0_ref, o1_ref, d0_ref, d1_ref, b_ref, w_ref, as_ref, ad_ref,
                   xl_ref, h_ref, asr_ref, adr_ref, m_ref):
    o = jnp.concatenate([o0_ref[...], o1_ref[...]], axis=1)
    den = d0_ref[...] + d1_ref[...] + 1e-16
    xl = jnp.maximum(o / den + b_ref[...], 0.0)
    xl_ref[...] = xl
    h = jnp.dot(xl, w_ref[...], preferred_element_type=jnp.float32)
    h_ref[0] = h[:, :F // 2]
    h_ref[1] = h[:, F // 2:]
    asr = jnp.sum(h * as_ref[...], axis=1, keepdims=True)
    adr = jnp.sum(h * ad_ref[...], axis=1, keepdims=True)
    asr_ref[...] = asr
    adr_ref[...] = adr
    m = jnp.maximum(jnp.max(asr) + jnp.max(adr), 0.0)
    m_ref[...] = jnp.full((8, 128), m, jnp.float32)


def _tc_final_body(x1_ref, x2_ref, o0_ref, o1_ref, d0_ref, d1_ref, b3_ref,
                   batch_ref, wl_ref, bl_ref, out_ref):
    o = jnp.concatenate([o0_ref[...], o1_ref[...]], axis=1)
    den = d0_ref[...] + d1_ref[...] + 1e-16
    x3 = jnp.maximum(o / den + b3_ref[...], 0.0)
    xc = jnp.concatenate([x1_ref[...], x2_ref[...], x3], axis=1)
    gid = lax.broadcasted_iota(jnp.int32, (N, G), 1)
    oh = (batch_ref[...] == gid).astype(jnp.float32)
    sums = lax.dot_general(oh, xc, (((0,), (0,)), ((), ())),
                           preferred_element_type=jnp.float32)
    ones = jnp.ones((N, 1), jnp.float32)
    counts = lax.dot_general(oh, ones, (((0,), (0,)), ((), ())),
                             preferred_element_type=jnp.float32)
    pooled = sums / jnp.maximum(counts, 1.0)
    logits = jnp.dot(pooled, wl_ref[...],
                     preferred_element_type=jnp.float32) + bl_ref[...]
    z = logits - jnp.max(logits, axis=1, keepdims=True)
    ez = jnp.exp(z)
    out_ref[...] = ez / jnp.sum(ez, axis=1, keepdims=True)


def _tc_prep1(x, w, a_s, a_d):
    return pl.pallas_call(
        _tc_prep1_body,
        out_shape=[
            jax.ShapeDtypeStruct((NC, N, F // 2), jnp.float32),
            jax.ShapeDtypeStruct((N, 1), jnp.float32),
            jax.ShapeDtypeStruct((N, 1), jnp.float32),
            jax.ShapeDtypeStruct((8, 128), jnp.float32),
        ],
    )(x, w, a_s.reshape(1, F), a_d.reshape(1, F))


def _tc_prep2(o0, o1, d0, d1, b, w, a_s, a_d):
    return pl.pallas_call(
        _tc_prep2_body,
        out_shape=[
            jax.ShapeDtypeStruct((N, F), jnp.float32),
            jax.ShapeDtypeStruct((NC, N, F // 2), jnp.float32),
            jax.ShapeDtypeStruct((N, 1), jnp.float32),
            jax.ShapeDtypeStruct((N, 1), jnp.float32),
            jax.ShapeDtypeStruct((8, 128), jnp.float32),
        ],
    )(o0, o1, d0.reshape(N, 1), d1.reshape(N, 1), b.reshape(1, F), w,
      a_s.reshape(1, F), a_d.reshape(1, F))


def _tc_final(x1, x2, o0, o1, d0, d1, b3, batch, wl, bl):
    return pl.pallas_call(
        _tc_final_body,
        out_shape=jax.ShapeDtypeStruct((G, OUT), jnp.float32),
    )(x1, x2, o0, o1, d0.reshape(N, 1), d1.reshape(N, 1), b3.reshape(1, F),
      batch.reshape(N, 1), wl, bl.reshape(1, OUT))


_SC_MESH = plsc.VectorSubcoreMesh(core_axis_name="c", subcore_axis_name="s")


@functools.partial(
    pl.kernel,
    out_type=[
        jax.ShapeDtypeStruct((NC, NP, F // 2), jnp.float32),
        jax.ShapeDtypeStruct((NC, NP), jnp.float32),
    ],
    mesh=_SC_MESH,
    compiler_params=pltpu.CompilerParams(
        needs_layout_passes=False, use_tc_tiling_on_sc=False),
    scratch_types=[
        pltpu.VMEM((NCHUNK, CHUNK), jnp.int32),
        pltpu.VMEM((NCHUNK, CHUNK), jnp.int32),
        pltpu.VMEM((NCHUNK, CHUNK), jnp.float32),
        pltpu.VMEM((N,), jnp.float32),
        pltpu.VMEM((N,), jnp.float32),
        pltpu.VMEM((CHUNK, F // 2), jnp.float32),
        pltpu.VMEM((CHUNK, F // 2), jnp.float32),
        pltpu.VMEM((RPT,), jnp.float32),
        pltpu.VMEM((16,), jnp.float32),
        pltpu.VMEM_SHARED((NP,), jnp.float32),
        pltpu.VMEM_SHARED((NP, F // 2), jnp.float32),
        pltpu.SemaphoreType.DMA,
        pltpu.SemaphoreType.DMA,
        pltpu.SemaphoreType.DMA,
    ],
)
def _sc_layer(h_hbm, as_hbm, ad_hbm, m_hbm, src_hbm, dst_hbm,
              out_hbm, den_hbm,
              src_v, dst_v, w_v, as_v, ad_v, rowbuf0, rowbuf1, zbuf, m_v,
              den_sh, out_sh, semd, sem0, sem1):
    sid = lax.axis_index("s")
    cid = lax.axis_index("c")

    pltpu.sync_copy(src_hbm.at[sid], src_v)
    pltpu.sync_copy(dst_hbm.at[sid], dst_v)
    pltpu.sync_copy(as_hbm, as_v)
    pltpu.sync_copy(ad_hbm, ad_v)
    pltpu.sync_copy(m_hbm.at[pl.ds(0, 16)], m_v)

    zero16 = jnp.zeros((16,), jnp.float32)

    def _zrow(r, carry):
        for c in range(F // 32):
            rowbuf0[r, pl.ds(c * 16, 16)] = zero16
        return carry
    lax.fori_loop(0, CHUNK, _zrow, 0)
    for k in range(RPT // CHUNK):
        pltpu.sync_copy(rowbuf0, out_sh.at[pl.ds(sid * RPT + k * CHUNK, CHUNK)])

    def _zden(r, carry):
        zbuf[pl.ds(r * 16, 16)] = zero16
        return carry
    lax.fori_loop(0, RPT // 16, _zden, 0)
    pltpu.sync_copy(zbuf, den_sh.at[pl.ds(sid * RPT, RPT)])

    plsc.subcore_barrier()

    m_vec = m_v[...]
    h_half = h_hbm.at[cid]
    dlo = cid * HALF

    def _gather(j, buf, s):
        return pltpu.make_async_copy(h_half.at[src_v.at[j]], buf, s)

    def _proc(j, buf):
        for g in range(CHUNK // 16):
            sv = src_v[j, pl.ds(g * 16, 16)]
            dv = dst_v[j, pl.ds(g * 16, 16)]
            e = plsc.load_gather(as_v, [sv]) + plsc.load_gather(ad_v, [dv])
            e = jnp.where(e >= 0.0, e, e * 0.2)
            w = jnp.exp(e - m_vec)
            w_v[j, pl.ds(g * 16, 16)] = w
            for i in range(16):
                a = w[i]
                r = g * 16 + i
                for c in range(F // 32):
                    buf[r, pl.ds(c * 16, 16)] = buf[r, pl.ds(c * 16, 16)] * a
        pltpu.sync_copy(buf, out_sh.at[dst_v.at[j]], add=True)

        @pl.when(jnp.logical_and(j >= dlo, j < dlo + HALF))
        def _():
            pltpu.async_copy(w_v.at[j], den_sh.at[dst_v.at[j]], semd, add=True)

    scope = jax.named_scope("fused_edge_pass")
    scope.__enter__()
    _gather(0, rowbuf0, sem0).start()

    def _p2(t, carry):
        j0 = 2 * t
        _gather(j0 + 1, rowbuf1, sem1).start()
        _gather(j0, rowbuf0, sem0).wait()
        _proc(j0, rowbuf0)

        @pl.when(t < NCHUNK // 2 - 1)
        def _():
            _gather(j0 + 2, rowbuf0, sem0).start()

        _gather(j0 + 1, rowbuf1, sem1).wait()
        _proc(j0 + 1, rowbuf1)
        return carry
    lax.fori_loop(0, NCHUNK // 2, _p2, 0)

    def _dd(jj, carry):
        j = dlo + jj
        pltpu.make_async_copy(w_v.at[j], den_sh.at[dst_v.at[j]], semd).wait()
        return carry
    lax.fori_loop(0, HALF, _dd, 0)
    scope.__exit__(None, None, None)

    plsc.subcore_barrier()

    pltpu.sync_copy(out_sh.at[pl.ds(sid * RPT, RPT)],
                    out_hbm.at[cid, pl.ds(sid * RPT, RPT)])
    pltpu.sync_copy(den_sh.at[pl.ds(sid * RPT, RPT)],
                    den_hbm.at[cid, pl.ds(sid * RPT, RPT)])


def kernel(x, edge_index, batch, W1, a1s, a1d, b1, W2, a2s, a2d, b2,
           W3, a3s, a3d, b3, Wl, bl):
    src3 = edge_index[0].reshape(NS, NCHUNK, CHUNK)
    dst3 = edge_index[1].reshape(NS, NCHUNK, CHUNK)

    h1, as1, ad1, m1 = _tc_prep1(x, W1, a1s, a1d)
    o1, den1 = _sc_layer(h1, as1.reshape(N), ad1.reshape(N), m1[0], src3, dst3)
    x1, h2, as2, ad2, m2 = _tc_prep2(o1[0, :N], o1[1, :N],
                                     den1[0, :N], den1[1, :N], b1, W2, a2s, a2d)
    o2, den2 = _sc_layer(h2, as2.reshape(N), ad2.reshape(N), m2[0], src3, dst3)
    x2, h3, as3, ad3, m3 = _tc_prep2(o2[0, :N], o2[1, :N],
                                     den2[0, :N], den2[1, :N], b2, W3, a3s, a3d)
    o3, den3 = _sc_layer(h3, as3.reshape(N), ad3.reshape(N), m3[0], src3, dst3)
    return _tc_final(x1, x2, o3[0, :N], o3[1, :N], den3[0, :N], den3[1, :N],
                     b3, batch, Wl, bl)


# speedup vs baseline: 53.2703x; 1.1701x over previous
import functools

import jax
import jax.numpy as jnp
from jax import lax
from jax.experimental import pallas as pl
from jax.experimental.pallas import tpu as pltpu
from jax.experimental.pallas import tpu_sc as plsc

N = 10000
E = 320000
D_IN = 128
F = 64
OUT = 10
G = 64

NC = 2
NS = 16
NP = 10240
RPT = NP // NS
EPT = E // NS
CHUNK = 80
NCHUNK = EPT // CHUNK
NQ = NCHUNK // 4


def _tc_prep1_body(x_ref, w_ref, as_ref, ad_ref, h_ref, asr_ref, adr_ref, m_ref):
    h = jnp.dot(x_ref[...], w_ref[...], preferred_element_type=jnp.float32)
    h_ref[0] = h[:, :F // 2]
    h_ref[1] = h[:, F // 2:]
    asr = jnp.sum(h * as_ref[...], axis=1, keepdims=True)
    adr = jnp.sum(h * ad_ref[...], axis=1, keepdims=True)
    asr_ref[...] = asr
    adr_ref[...] = adr
    m = jnp.maximum(jnp.max(asr) + jnp.max(adr), 0.0)
    m_ref[...] = jnp.full((8, 128), m, jnp.float32)


def _tc_prep2_body(o_ref, b_ref, w_ref, as_ref, ad_ref,
                   xl_ref, h_ref, asr_ref, adr_ref, m_ref):
    o = jnp.concatenate([o_ref[0, :N, :], o_ref[1, :N, :]], axis=1)
    xl = jnp.maximum(o + b_ref[...], 0.0)
    xl_ref[...] = xl
    h = jnp.dot(xl, w_ref[...], preferred_element_type=jnp.float32)
    h_ref[0] = h[:, :F // 2]
    h_ref[1] = h[:, F // 2:]
    asr = jnp.sum(h * as_ref[...], axis=1, keepdims=True)
    adr = jnp.sum(h * ad_ref[...], axis=1, keepdims=True)
    asr_ref[...] = asr
    adr_ref[...] = adr
    m = jnp.maximum(jnp.max(asr) + jnp.max(adr), 0.0)
    m_ref[...] = jnp.full((8, 128), m, jnp.float32)


def _tc_final_body(x1_ref, x2_ref, o_ref, b3_ref, batch_ref,
                   wl_ref, bl_ref, out_ref):
    o = jnp.concatenate([o_ref[0, :N, :], o_ref[1, :N, :]], axis=1)
    x3 = jnp.maximum(o + b3_ref[...], 0.0)
    xc = jnp.concatenate([x1_ref[...], x2_ref[...], x3], axis=1)
    gid = lax.broadcasted_iota(jnp.int32, (N, G), 1)
    oh = (batch_ref[...] == gid).astype(jnp.float32)
    sums = lax.dot_general(oh, xc, (((0,), (0,)), ((), ())),
                           preferred_element_type=jnp.float32)
    ones = jnp.ones((N, 1), jnp.float32)
    counts = lax.dot_general(oh, ones, (((0,), (0,)), ((), ())),
                             preferred_element_type=jnp.float32)
    pooled = sums / jnp.maximum(counts, 1.0)
    logits = jnp.dot(pooled, wl_ref[...],
                     preferred_element_type=jnp.float32) + bl_ref[...]
    z = logits - jnp.max(logits, axis=1, keepdims=True)
    ez = jnp.exp(z)
    out_ref[...] = ez / jnp.sum(ez, axis=1, keepdims=True)


def _tc_prep1(x, w, a_s, a_d):
    return pl.pallas_call(
        _tc_prep1_body,
        out_shape=[
            jax.ShapeDtypeStruct((NC, N, F // 2), jnp.float32),
            jax.ShapeDtypeStruct((N, 1), jnp.float32),
            jax.ShapeDtypeStruct((N, 1), jnp.float32),
            jax.ShapeDtypeStruct((8, 128), jnp.float32),
        ],
    )(x, w, a_s.reshape(1, F), a_d.reshape(1, F))


def _tc_prep2(o, b, w, a_s, a_d):
    return pl.pallas_call(
        _tc_prep2_body,
        out_shape=[
            jax.ShapeDtypeStruct((N, F), jnp.float32),
            jax.ShapeDtypeStruct((NC, N, F // 2), jnp.float32),
            jax.ShapeDtypeStruct((N, 1), jnp.float32),
            jax.ShapeDtypeStruct((N, 1), jnp.float32),
            jax.ShapeDtypeStruct((8, 128), jnp.float32),
        ],
    )(o, b.reshape(1, F), w, a_s.reshape(1, F), a_d.reshape(1, F))


def _tc_final(x1, x2, o, b3, batch, wl, bl):
    return pl.pallas_call(
        _tc_final_body,
        out_shape=jax.ShapeDtypeStruct((G, OUT), jnp.float32),
    )(x1, x2, o, b3.reshape(1, F), batch.reshape(N, 1), wl,
      bl.reshape(1, OUT))


_SC_MESH = plsc.VectorSubcoreMesh(core_axis_name="c", subcore_axis_name="s")


@functools.partial(
    pl.kernel,
    out_type=jax.ShapeDtypeStruct((NC, NP, F // 2), jnp.float32),
    mesh=_SC_MESH,
    compiler_params=pltpu.CompilerParams(
        needs_layout_passes=False, use_tc_tiling_on_sc=False),
    scratch_types=[
        pltpu.VMEM((NCHUNK, CHUNK), jnp.int32),
        pltpu.VMEM((NCHUNK, CHUNK), jnp.int32),
        pltpu.VMEM((NCHUNK, CHUNK), jnp.float32),
        pltpu.VMEM((N,), jnp.float32),
        pltpu.VMEM((N,), jnp.float32),
        pltpu.VMEM((CHUNK, F // 2), jnp.float32),
        pltpu.VMEM((CHUNK, F // 2), jnp.float32),
        pltpu.VMEM((CHUNK, F // 2), jnp.float32),
        pltpu.VMEM((CHUNK, F // 2), jnp.float32),
        pltpu.VMEM((RPT,), jnp.float32),
        pltpu.VMEM((16,), jnp.float32),
        pltpu.VMEM_SHARED((NP,), jnp.float32),
        pltpu.VMEM_SHARED((NP, F // 2), jnp.float32),
        pltpu.SemaphoreType.DMA,
        pltpu.SemaphoreType.DMA,
        pltpu.SemaphoreType.DMA,
        pltpu.SemaphoreType.DMA,
        pltpu.SemaphoreType.DMA,
        pltpu.SemaphoreType.DMA,
        pltpu.SemaphoreType.DMA,
        pltpu.SemaphoreType.DMA,
        pltpu.SemaphoreType.DMA,
    ],
)
def _sc_layer(h_hbm, as_hbm, ad_hbm, m_hbm, src_hbm, dst_hbm, out_hbm,
              src_v, dst_v, w_v, as_v, ad_v, rb0, rb1, rb2, rb3, dbuf, m_v,
              den_sh, out_sh, semd, g0, g1, g2, g3, s0, s1, s2, s3):
    sid = lax.axis_index("s")
    cid = lax.axis_index("c")
    bufs = (rb0, rb1, rb2, rb3)
    gsems = (g0, g1, g2, g3)
    ssems = (s0, s1, s2, s3)

    pltpu.sync_copy(src_hbm.at[sid], src_v)
    pltpu.sync_copy(dst_hbm.at[sid], dst_v)
    pltpu.sync_copy(as_hbm, as_v)
    pltpu.sync_copy(ad_hbm, ad_v)
    pltpu.sync_copy(m_hbm.at[0, pl.ds(0, 16)], m_v)

    zero16 = jnp.zeros((16,), jnp.float32)
    zero16i = jnp.zeros((16,), jnp.int32)

    def _zrow(r, carry):
        for c in range(F // 32):
            rb0[r, pl.ds(c * 16, 16)] = zero16
        return carry
    lax.fori_loop(0, CHUNK, _zrow, 0)
    for k in range(RPT // CHUNK):
        pltpu.sync_copy(rb0, out_sh.at[pl.ds(sid * RPT + k * CHUNK, CHUNK)])

    def _zden(r, carry):
        dbuf[pl.ds(r * 16, 16)] = zero16
        return carry
    lax.fori_loop(0, RPT // 16, _zden, 0)
    pltpu.sync_copy(dbuf, den_sh.at[pl.ds(sid * RPT, RPT)])

    plsc.subcore_barrier()

    m_vec = m_v[...]
    h_half = h_hbm.at[cid]

    def _gat(j, b):
        return pltpu.make_async_copy(h_half.at[src_v.at[j]], bufs[b], gsems[b])

    def _sct(j, b):
        return pltpu.make_async_copy(bufs[b], out_sh.at[dst_v.at[j]], ssems[b])

    def _proc(j, b):
        buf = bufs[b]
        for g in range(CHUNK // 16):
            sv = src_v[j, pl.ds(g * 16, 16)]
            dv = dst_v[j, pl.ds(g * 16, 16)]
            e = plsc.load_gather(as_v, [sv]) + plsc.load_gather(ad_v, [dv])
            e = jnp.where(e >= 0.0, e, e * 0.2)
            w = jnp.exp(e - m_vec)
            w_v[j, pl.ds(g * 16, 16)] = w
            for i in range(16):
                a = w[i]
                r = g * 16 + i
                for c in range(F // 32):
                    buf[r, pl.ds(c * 16, 16)] = buf[r, pl.ds(c * 16, 16)] * a
        _sct(j, b).start(add=True)
        pltpu.async_copy(w_v.at[j], den_sh.at[dst_v.at[j]], semd, add=True)

    def _step(j, b):
        _gat(j, b).wait()
        _proc(j, b)
        b2 = (b + 2) % 4

        @pl.when(j >= 2)
        def _():
            _sct(j - 2, b2).wait()

        @pl.when(j + 2 < NCHUNK)
        def _():
            _gat(j + 2, b2).start()

    scope = jax.named_scope("fused_edge_pass")
    scope.__enter__()
    _gat(0, 0).start()
    _gat(1, 1).start()

    def _ring(t, carry):
        j0 = 4 * t
        for b in range(4):
            _step(j0 + b, b)
        return carry
    lax.fori_loop(0, NQ, _ring, 0)
    _step(jnp.int32(NCHUNK - 2), 0)
    _step(jnp.int32(NCHUNK - 1), 1)
    _sct(NCHUNK - 2, 0).wait()
    _sct(NCHUNK - 1, 1).wait()

    def _dd(j, carry):
        pltpu.make_async_copy(w_v.at[j], den_sh.at[dst_v.at[j]], semd).wait()
        return carry
    lax.fori_loop(0, NCHUNK, _dd, 0)
    scope.__exit__(None, None, None)

    plsc.subcore_barrier()

    pltpu.sync_copy(den_sh.at[pl.ds(sid * RPT, RPT)], dbuf)
    one16 = jnp.ones((16,), jnp.float32)

    for k in range(RPT // CHUNK):
        base = sid * RPT + k * CHUNK
        pltpu.sync_copy(out_sh.at[pl.ds(base, CHUNK)], rb0)
        for g in range(CHUNK // 16):
            den = dbuf[pl.ds(k * CHUNK + g * 16, 16)]
            rden = one16 / (den + 1e-16)
            for i in range(16):
                a = rden[i]
                r = g * 16 + i
                for c in range(F // 32):
                    rb0[r, pl.ds(c * 16, 16)] = rb0[r, pl.ds(c * 16, 16)] * a
        pltpu.sync_copy(rb0, out_hbm.at[cid, pl.ds(base, CHUNK)])


def kernel(x, edge_index, batch, W1, a1s, a1d, b1, W2, a2s, a2d, b2,
           W3, a3s, a3d, b3, Wl, bl):
    src3 = edge_index[0].reshape(NS, NCHUNK, CHUNK)
    dst3 = edge_index[1].reshape(NS, NCHUNK, CHUNK)

    h1, as1, ad1, m1 = _tc_prep1(x, W1, a1s, a1d)
    o1 = _sc_layer(h1, as1.reshape(N), ad1.reshape(N), m1, src3, dst3)
    x1, h2, as2, ad2, m2 = _tc_prep2(o1, b1, W2, a2s, a2d)
    o2 = _sc_layer(h2, as2.reshape(N), ad2.reshape(N), m2, src3, dst3)
    x2, h3, as3, ad3, m3 = _tc_prep2(o2, b2, W3, a3s, a3d)
    o3 = _sc_layer(h3, as3.reshape(N), ad3.reshape(N), m3, src3, dst3)
    return _tc_final(x1, x2, o3, b3, batch, Wl, bl)
